# baseline (speedup 1.0000x reference)
import functools

import jax
import jax.numpy as jnp
from jax import lax
from jax.experimental import pallas as pl
from jax.experimental.pallas import tpu as pltpu
from jax.experimental.pallas import tpu_sc as plsc

D_MODEL = 1024
NC = 2
NS = 16
NW = NC * NS
LANES = 16


def _sc_time_gather(board2, time_emb):
    B = board2.shape[0]
    bpw = B // NW
    mesh = plsc.VectorSubcoreMesh(core_axis_name="c", subcore_axis_name="s")

    CS = 8
    nh = bpw // LANES
    ncs = bpw // CS

    @functools.partial(
        pl.kernel,
        mesh=mesh,
        out_type=jax.ShapeDtypeStruct((B, D_MODEL), jnp.float32),
        scratch_types=[
            pltpu.VMEM((bpw, 128), jnp.float32),
            pltpu.VMEM((nh, LANES), jnp.int32),
            pltpu.VMEM((bpw, D_MODEL), jnp.float32),
            pltpu.SemaphoreType.DMA,
            pltpu.SemaphoreType.DMA,
        ],
    )
    def k(board_hbm, time_hbm, out_hbm, board_v, idx_v, rows_v, gsem, osem):
        wid = lax.axis_index("s") * NC + lax.axis_index("c")
        base = wid * bpw
        pltpu.sync_copy(board_hbm.at[pl.ds(base, bpw)], board_v)
        lane = lax.iota(jnp.int32, LANES)

        def idx_half(h):
            def body(b, vec):
                row = h * LANES + b
                v0 = board_v[row, pl.ds(0, LANES)] + board_v[row, pl.ds(LANES, LANES)]
                v1 = board_v[row, pl.ds(2 * LANES, LANES)] + board_v[row, pl.ds(3 * LANES, LANES)]
                v2 = board_v[row, pl.ds(4 * LANES, LANES)] + board_v[row, pl.ds(5 * LANES, LANES)]
                v3 = board_v[row, pl.ds(6 * LANES, LANES)] + board_v[row, pl.ds(7 * LANES, LANES)]
                acc = (v0 + v1) + (v2 + v3)
                ss = [acc[l] for l in range(LANES)]
                while len(ss) > 1:
                    ss = [a + b2 for a, b2 in zip(ss[::2], ss[1::2])]
                s = ss[0]
                i0 = s.astype(jnp.int32)
                i0 = jnp.where(i0.astype(jnp.float32) > s, i0 - 1, i0)
                idxb = jnp.maximum(i0 - 4, 0)
                return jnp.where(lane == b, idxb, vec)

            idx_v[h, pl.ds(0, LANES)] = lax.fori_loop(
                0, LANES, body, jnp.zeros((LANES,), jnp.int32)
            )

        for h in range(nh):
            idx_half(h)
        gathers = []
        for q in range(ncs):
            h, o = divmod(q * CS, LANES)
            gathers.append(
                pltpu.async_copy(
                    time_hbm.at[idx_v.at[h, pl.ds(o, CS)]],
                    rows_v.at[pl.ds(q * CS, CS)],
                    gsem,
                )
            )
        outs = []
        for q in range(ncs):
            gathers[q].wait()
            outs.append(
                pltpu.async_copy(
                    rows_v.at[pl.ds(q * CS, CS)],
                    out_hbm.at[pl.ds(base + q * CS, CS)],
                    osem,
                )
            )
        for o in outs:
            o.wait()

    return k(board2, time_emb)


def _tc_add(x4, row_emb, col_emb, t_full, bb):
    B = x4.shape[0]

    def body(x_ref, r_ref, c_ref, t_ref, o_ref):
        pos = r_ref[:][None, :, None, :] + c_ref[:][None, None, :, :]
        o_ref[:] = x_ref[:] + (pos + t_ref[:][:, None, None, :])

    return pl.pallas_call(
        body,
        grid=(B // bb,),
        in_specs=[
            pl.BlockSpec((bb, 8, 8, D_MODEL), lambda i: (i, 0, 0, 0)),
            pl.BlockSpec((8, D_MODEL), lambda i: (0, 0)),
            pl.BlockSpec((8, D_MODEL), lambda i: (0, 0)),
            pl.BlockSpec((bb, D_MODEL), lambda i: (i, 0)),
        ],
        out_specs=pl.BlockSpec((bb, 8, 8, D_MODEL), lambda i: (i, 0, 0, 0)),
        out_shape=jax.ShapeDtypeStruct(x4.shape, x4.dtype),
    )(x4, row_emb, col_emb, t_full)


def _sc_time_gather_chunk(board2, time_emb, chunk_off, cb):
    bpw = cb // NW
    mesh = plsc.VectorSubcoreMesh(core_axis_name="c", subcore_axis_name="s")
    nh = bpw // LANES

    @functools.partial(
        pl.kernel,
        mesh=mesh,
        out_type=jax.ShapeDtypeStruct((cb, D_MODEL), jnp.float32),
        scratch_types=[
            pltpu.VMEM((bpw, 128), jnp.float32),
            pltpu.VMEM((nh, LANES), jnp.int32),
            pltpu.VMEM((bpw, D_MODEL), jnp.float32),
            pltpu.SemaphoreType.DMA,
            pltpu.SemaphoreType.DMA,
        ],
    )
    def k(board_hbm, time_hbm, out_hbm, board_v, idx_v, rows_v, gsem, osem):
        wid = lax.axis_index("s") * NC + lax.axis_index("c")
        base = wid * bpw
        pltpu.sync_copy(board_hbm.at[pl.ds(chunk_off + base, bpw)], board_v)
        lane = lax.iota(jnp.int32, LANES)

        def idx_half(h):
            def body(b, vec):
                row = h * LANES + b
                v0 = board_v[row, pl.ds(0, LANES)] + board_v[row, pl.ds(LANES, LANES)]
                v1 = board_v[row, pl.ds(2 * LANES, LANES)] + board_v[row, pl.ds(3 * LANES, LANES)]
                v2 = board_v[row, pl.ds(4 * LANES, LANES)] + board_v[row, pl.ds(5 * LANES, LANES)]
                v3 = board_v[row, pl.ds(6 * LANES, LANES)] + board_v[row, pl.ds(7 * LANES, LANES)]
                acc = (v0 + v1) + (v2 + v3)
                ss = [acc[l] for l in range(LANES)]
                while len(ss) > 1:
                    ss = [a + b2 for a, b2 in zip(ss[::2], ss[1::2])]
                s = ss[0]
                i0 = s.astype(jnp.int32)
                i0 = jnp.where(i0.astype(jnp.float32) > s, i0 - 1, i0)
                idxb = jnp.maximum(i0 - 4, 0)
                return jnp.where(lane == b, idxb, vec)

            idx_v[h, pl.ds(0, LANES)] = lax.fori_loop(
                0, LANES, body, jnp.zeros((LANES,), jnp.int32)
            )

        gathers = []
        for h in range(nh):
            idx_half(h)
            gathers.append(
                pltpu.async_copy(
                    time_hbm.at[idx_v.at[h]],
                    rows_v.at[pl.ds(h * LANES, LANES)],
                    gsem,
                )
            )
        outs = []
        for h in range(nh):
            gathers[h].wait()
            outs.append(
                pltpu.async_copy(
                    rows_v.at[pl.ds(h * LANES, LANES)],
                    out_hbm.at[pl.ds(base + h * LANES, LANES)],
                    osem,
                )
            )
        for o in outs:
            o.wait()

    return k(board2, time_emb)


def _tc_add_chunk(x4, row_emb, col_emb, t_chunk, prev, chunk_idx, cb, bb):
    blk0 = chunk_idx * cb // bb

    def body(x_ref, r_ref, c_ref, t_ref, *rest):
        o_ref = rest[-1]
        pos = r_ref[:][None, :, None, :] + c_ref[:][None, None, :, :]
        o_ref[:] = x_ref[:] + (pos + t_ref[:][:, None, None, :])

    in_specs = [
        pl.BlockSpec((bb, 8, 8, D_MODEL), lambda i: (i + blk0, 0, 0, 0)),
        pl.BlockSpec((8, D_MODEL), lambda i: (0, 0)),
        pl.BlockSpec((8, D_MODEL), lambda i: (0, 0)),
        pl.BlockSpec((bb, D_MODEL), lambda i: (i, 0)),
    ]
    args = [x4, row_emb, col_emb, t_chunk]
    aliases = {}
    if prev is not None:
        in_specs.append(pl.BlockSpec(memory_space=pl.ANY))
        args.append(prev)
        aliases = {4: 0}
    return pl.pallas_call(
        body,
        grid=(cb // bb,),
        in_specs=in_specs,
        out_specs=pl.BlockSpec((bb, 8, 8, D_MODEL), lambda i: (i + blk0, 0, 0, 0)),
        out_shape=jax.ShapeDtypeStruct(x4.shape, x4.dtype),
        input_output_aliases=aliases,
    )(*args)


def kernel(x, board, row_emb, col_emb, time_emb):
    B = x.shape[0]
    board2 = board.reshape(B, 128)
    x4 = x.reshape(B, 8, 8, D_MODEL)
    cb = B // 2
    t0 = _sc_time_gather_chunk(board2, time_emb, 0, cb)
    t1 = _sc_time_gather_chunk(board2, time_emb, cb, cb)
    o0 = _tc_add_chunk(x4, row_emb, col_emb, t0, None, 0, cb, bb=32)
    o1 = _tc_add_chunk(x4, row_emb, col_emb, t1, o0, 1, cb, bb=32)
    return o1.reshape(x.shape)


# speedup vs baseline: 1.0029x; 1.0029x over previous
import functools

import jax
import jax.numpy as jnp
from jax import lax
from jax.experimental import pallas as pl
from jax.experimental.pallas import tpu as pltpu
from jax.experimental.pallas import tpu_sc as plsc

D_MODEL = 1024
NC = 2
NS = 16
NW = NC * NS
LANES = 16


def _sc_time_gather(board2, time_emb):
    B = board2.shape[0]
    bpw = B // NW
    mesh = plsc.VectorSubcoreMesh(core_axis_name="c", subcore_axis_name="s")

    CS = 8
    nh = bpw // LANES
    ncs = bpw // CS

    @functools.partial(
        pl.kernel,
        mesh=mesh,
        out_type=jax.ShapeDtypeStruct((B, D_MODEL), jnp.float32),
        scratch_types=[
            pltpu.VMEM((bpw, 128), jnp.float32),
            pltpu.VMEM((nh, LANES), jnp.int32),
            pltpu.VMEM((bpw, D_MODEL), jnp.float32),
            pltpu.SemaphoreType.DMA,
            pltpu.SemaphoreType.DMA,
        ],
    )
    def k(board_hbm, time_hbm, out_hbm, board_v, idx_v, rows_v, gsem, osem):
        wid = lax.axis_index("s") * NC + lax.axis_index("c")
        base = wid * bpw
        pltpu.sync_copy(board_hbm.at[pl.ds(base, bpw)], board_v)
        lane = lax.iota(jnp.int32, LANES)

        def idx_half(h):
            def body(b, vec):
                row = h * LANES + b
                v0 = board_v[row, pl.ds(0, LANES)] + board_v[row, pl.ds(LANES, LANES)]
                v1 = board_v[row, pl.ds(2 * LANES, LANES)] + board_v[row, pl.ds(3 * LANES, LANES)]
                v2 = board_v[row, pl.ds(4 * LANES, LANES)] + board_v[row, pl.ds(5 * LANES, LANES)]
                v3 = board_v[row, pl.ds(6 * LANES, LANES)] + board_v[row, pl.ds(7 * LANES, LANES)]
                acc = (v0 + v1) + (v2 + v3)
                ss = [acc[l] for l in range(LANES)]
                while len(ss) > 1:
                    ss = [a + b2 for a, b2 in zip(ss[::2], ss[1::2])]
                s = ss[0]
                i0 = s.astype(jnp.int32)
                i0 = jnp.where(i0.astype(jnp.float32) > s, i0 - 1, i0)
                idxb = jnp.maximum(i0 - 4, 0)
                return jnp.where(lane == b, idxb, vec)

            idx_v[h, pl.ds(0, LANES)] = lax.fori_loop(
                0, LANES, body, jnp.zeros((LANES,), jnp.int32)
            )

        for h in range(nh):
            idx_half(h)
        gathers = []
        for q in range(ncs):
            h, o = divmod(q * CS, LANES)
            gathers.append(
                pltpu.async_copy(
                    time_hbm.at[idx_v.at[h, pl.ds(o, CS)]],
                    rows_v.at[pl.ds(q * CS, CS)],
                    gsem,
                )
            )
        outs = []
        for q in range(ncs):
            gathers[q].wait()
            outs.append(
                pltpu.async_copy(
                    rows_v.at[pl.ds(q * CS, CS)],
                    out_hbm.at[pl.ds(base + q * CS, CS)],
                    osem,
                )
            )
        for o in outs:
            o.wait()

    return k(board2, time_emb)


def _tc_add(x4, row_emb, col_emb, t_full, bb):
    B = x4.shape[0]

    def body(x_ref, r_ref, c_ref, t_ref, o_ref):
        pos = r_ref[:][None, :, None, :] + c_ref[:][None, None, :, :]
        o_ref[:] = x_ref[:] + (pos + t_ref[:][:, None, None, :])

    return pl.pallas_call(
        body,
        grid=(B // bb,),
        in_specs=[
            pl.BlockSpec((bb, 8, 8, D_MODEL), lambda i: (i, 0, 0, 0)),
            pl.BlockSpec((8, D_MODEL), lambda i: (0, 0)),
            pl.BlockSpec((8, D_MODEL), lambda i: (0, 0)),
            pl.BlockSpec((bb, D_MODEL), lambda i: (i, 0)),
        ],
        out_specs=pl.BlockSpec((bb, 8, 8, D_MODEL), lambda i: (i, 0, 0, 0)),
        out_shape=jax.ShapeDtypeStruct(x4.shape, x4.dtype),
    )(x4, row_emb, col_emb, t_full)


def _sc_time_gather_chunk(board2, time_emb):
    cb = board2.shape[0]
    bpw = cb // NW
    mesh = plsc.VectorSubcoreMesh(core_axis_name="c", subcore_axis_name="s")
    groups = []
    r = bpw
    while r > 0:
        groups.append(min(r, LANES))
        r -= LANES
    nh = len(groups)

    @functools.partial(
        pl.kernel,
        mesh=mesh,
        out_type=jax.ShapeDtypeStruct((cb, D_MODEL), jnp.float32),
        scratch_types=[
            pltpu.VMEM((bpw, 128), jnp.float32),
            pltpu.VMEM((nh, LANES), jnp.int32),
            pltpu.VMEM((bpw, D_MODEL), jnp.float32),
            pltpu.SemaphoreType.DMA,
            pltpu.SemaphoreType.DMA,
        ],
    )
    def k(board_hbm, time_hbm, out_hbm, board_v, idx_v, rows_v, gsem, osem):
        wid = lax.axis_index("s") * NC + lax.axis_index("c")
        base = wid * bpw
        pltpu.sync_copy(board_hbm.at[pl.ds(base, bpw)], board_v)
        lane = lax.iota(jnp.int32, LANES)

        def idx_group(h, cnt):
            def body(b, vec):
                row = h * LANES + b
                v0 = board_v[row, pl.ds(0, LANES)] + board_v[row, pl.ds(LANES, LANES)]
                v1 = board_v[row, pl.ds(2 * LANES, LANES)] + board_v[row, pl.ds(3 * LANES, LANES)]
                v2 = board_v[row, pl.ds(4 * LANES, LANES)] + board_v[row, pl.ds(5 * LANES, LANES)]
                v3 = board_v[row, pl.ds(6 * LANES, LANES)] + board_v[row, pl.ds(7 * LANES, LANES)]
                acc = (v0 + v1) + (v2 + v3)
                ss = [acc[l] for l in range(LANES)]
                while len(ss) > 1:
                    ss = [a + b2 for a, b2 in zip(ss[::2], ss[1::2])]
                s = ss[0]
                i0 = s.astype(jnp.int32)
                i0 = jnp.where(i0.astype(jnp.float32) > s, i0 - 1, i0)
                idxb = jnp.maximum(i0 - 4, 0)
                return jnp.where(lane == b, idxb, vec)

            idx_v[h, pl.ds(0, LANES)] = lax.fori_loop(
                0, cnt, body, jnp.zeros((LANES,), jnp.int32)
            )

        gathers = []
        for h, cnt in enumerate(groups):
            idx_group(h, cnt)
            gathers.append(
                pltpu.async_copy(
                    time_hbm.at[idx_v.at[h, pl.ds(0, cnt)]],
                    rows_v.at[pl.ds(h * LANES, cnt)],
                    gsem,
                )
            )
        outs = []
        for h, cnt in enumerate(groups):
            gathers[h].wait()
            outs.append(
                pltpu.async_copy(
                    rows_v.at[pl.ds(h * LANES, cnt)],
                    out_hbm.at[pl.ds(base + h * LANES, cnt)],
                    osem,
                )
            )
        for o in outs:
            o.wait()

    return k(board2, time_emb)


def _tc_add_chunk(x4, row_emb, col_emb, t_chunk, prev, blk0, cb, bb):
    def body(x_ref, r_ref, c_ref, t_ref, *rest):
        o_ref = rest[-1]
        pos = r_ref[:][None, :, None, :] + c_ref[:][None, None, :, :]
        o_ref[:] = x_ref[:] + (pos + t_ref[:][:, None, None, :])

    in_specs = [
        pl.BlockSpec((bb, 8, 8, D_MODEL), lambda i: (i + blk0, 0, 0, 0)),
        pl.BlockSpec((8, D_MODEL), lambda i: (0, 0)),
        pl.BlockSpec((8, D_MODEL), lambda i: (0, 0)),
        pl.BlockSpec((bb, D_MODEL), lambda i: (i, 0)),
    ]
    args = [x4, row_emb, col_emb, t_chunk]
    aliases = {}
    if prev is not None:
        in_specs.append(pl.BlockSpec(memory_space=pl.ANY))
        args.append(prev)
        aliases = {4: 0}
    return pl.pallas_call(
        body,
        grid=(cb // bb,),
        in_specs=in_specs,
        out_specs=pl.BlockSpec((bb, 8, 8, D_MODEL), lambda i: (i + blk0, 0, 0, 0)),
        out_shape=jax.ShapeDtypeStruct(x4.shape, x4.dtype),
        input_output_aliases=aliases,
    )(*args)


def kernel(x, board, row_emb, col_emb, time_emb):
    B = x.shape[0]
    x4 = x.reshape(B, 8, 8, D_MODEL)
    cb0 = 256
    cb1 = B - cb0
    b20 = board[:cb0].reshape(cb0, 128)
    b21 = board[cb0:].reshape(cb1, 128)
    t0 = _sc_time_gather_chunk(b20, time_emb)
    t1 = _sc_time_gather_chunk(b21, time_emb)
    o0 = _tc_add_chunk(x4, row_emb, col_emb, t0, None, 0, cb0, bb=32)
    o1 = _tc_add_chunk(x4, row_emb, col_emb, t1, o0, cb0 // 32, cb1, bb=32)
    return o1.reshape(x.shape)


# speedup vs baseline: 1.0033x; 1.0004x over previous
import functools

import jax
import jax.numpy as jnp
from jax import lax
from jax.experimental import pallas as pl
from jax.experimental.pallas import tpu as pltpu
from jax.experimental.pallas import tpu_sc as plsc

D_MODEL = 1024
NC = 2
NS = 16
NW = NC * NS
LANES = 16


def _sc_time_gather(board2, time_emb):
    B = board2.shape[0]
    bpw = B // NW
    mesh = plsc.VectorSubcoreMesh(core_axis_name="c", subcore_axis_name="s")
    groups = []
    r = bpw
    while r > 0:
        groups.append(min(r, LANES))
        r -= LANES
    nh = len(groups)

    @functools.partial(
        pl.kernel,
        mesh=mesh,
        out_type=jax.ShapeDtypeStruct((B, D_MODEL), jnp.float32),
        scratch_types=[
            pltpu.VMEM((bpw, 128), jnp.float32),
            pltpu.VMEM((nh, LANES), jnp.int32),
            pltpu.VMEM((bpw, D_MODEL), jnp.float32),
            pltpu.SemaphoreType.DMA,
            pltpu.SemaphoreType.DMA,
        ],
    )
    def k(board_hbm, time_hbm, out_hbm, board_v, idx_v, rows_v, gsem, osem):
        wid = lax.axis_index("s") * NC + lax.axis_index("c")
        base = wid * bpw
        pltpu.sync_copy(board_hbm.at[pl.ds(base, bpw)], board_v)
        lane = lax.iota(jnp.int32, LANES)

        def idx_group(h, cnt):
            def body(b, vec):
                row = h * LANES + b
                v0 = board_v[row, pl.ds(0, LANES)] + board_v[row, pl.ds(LANES, LANES)]
                v1 = board_v[row, pl.ds(2 * LANES, LANES)] + board_v[row, pl.ds(3 * LANES, LANES)]
                v2 = board_v[row, pl.ds(4 * LANES, LANES)] + board_v[row, pl.ds(5 * LANES, LANES)]
                v3 = board_v[row, pl.ds(6 * LANES, LANES)] + board_v[row, pl.ds(7 * LANES, LANES)]
                acc = (v0 + v1) + (v2 + v3)
                ss = [acc[l] for l in range(LANES)]
                while len(ss) > 1:
                    ss = [a + b2 for a, b2 in zip(ss[::2], ss[1::2])]
                s = ss[0]
                i0 = s.astype(jnp.int32)
                i0 = jnp.where(i0.astype(jnp.float32) > s, i0 - 1, i0)
                idxb = jnp.maximum(i0 - 4, 0)
                return jnp.where(lane == b, idxb, vec)

            idx_v[h, pl.ds(0, LANES)] = lax.fori_loop(
                0, cnt, body, jnp.zeros((LANES,), jnp.int32)
            )

        gathers = []
        for h, cnt in enumerate(groups):
            idx_group(h, cnt)
            gathers.append(
                pltpu.async_copy(
                    time_hbm.at[idx_v.at[h, pl.ds(0, cnt)]],
                    rows_v.at[pl.ds(h * LANES, cnt)],
                    gsem,
                )
            )
        outs = []
        for h, cnt in enumerate(groups):
            gathers[h].wait()
            outs.append(
                pltpu.async_copy(
                    rows_v.at[pl.ds(h * LANES, cnt)],
                    out_hbm.at[pl.ds(base + h * LANES, cnt)],
                    osem,
                )
            )
        for o in outs:
            o.wait()

    return k(board2, time_emb)


def _tc_add(x4, row_emb, col_emb, t_full, bb):
    B = x4.shape[0]

    def body(x_ref, r_ref, c_ref, t_ref, o_ref):
        pos = r_ref[:][None, :, None, :] + c_ref[:][None, None, :, :]
        o_ref[:] = x_ref[:] + (pos + t_ref[:][:, None, None, :])

    return pl.pallas_call(
        body,
        grid=(B // bb,),
        in_specs=[
            pl.BlockSpec((bb, 8, 8, D_MODEL), lambda i: (i, 0, 0, 0)),
            pl.BlockSpec((8, D_MODEL), lambda i: (0, 0)),
            pl.BlockSpec((8, D_MODEL), lambda i: (0, 0)),
            pl.BlockSpec((bb, D_MODEL), lambda i: (i, 0)),
        ],
        out_specs=pl.BlockSpec((bb, 8, 8, D_MODEL), lambda i: (i, 0, 0, 0)),
        out_shape=jax.ShapeDtypeStruct(x4.shape, x4.dtype),
    )(x4, row_emb, col_emb, t_full)


def kernel(x, board, row_emb, col_emb, time_emb):
    B = x.shape[0]
    board2 = board.reshape(B, 128)
    t_full = _sc_time_gather(board2, time_emb)
    x4 = x.reshape(B, 8, 8, D_MODEL)
    out4 = _tc_add(x4, row_emb, col_emb, t_full, bb=32)
    return out4.reshape(x.shape)
